# baseline (speedup 1.0000x reference)
import jax
import jax.numpy as jnp
from jax import lax
from jax.experimental import pallas as pl
from jax.experimental.pallas import tpu as pltpu
from jax.experimental.pallas import tpu_sc as plsc

NC = 2
NS = 16
L = 16
NW = NC * NS


def _sc_body(bpw, vi_hbm, vj_hbm, ft_hbm, st_hbm, ct_hbm, out1_hbm, out2_hbm,
             idx_i, idx_j, ra, rb, rc, rd, o1, o2, sem):
    wid = lax.axis_index("s") * NC + lax.axis_index("c")
    base = wid * bpw
    pltpu.sync_copy(vi_hbm.at[pl.ds(base, bpw)], idx_i)
    pltpu.sync_copy(vj_hbm.at[pl.ds(base, bpw)], idx_j)
    ca = pltpu.async_copy(ft_hbm.at[idx_i], ra, sem)
    cb = pltpu.async_copy(ft_hbm.at[idx_j], rb, sem)
    cc = pltpu.async_copy(st_hbm.at[idx_i], rc, sem)
    cd = pltpu.async_copy(ct_hbm.at[idx_j], rd, sem)
    ca.wait()
    cb.wait()
    cc.wait()
    cd.wait()

    lanes = lax.iota(jnp.int32, L)

    def group(g, carry):
        rows = lanes + g * L
        acc1 = jnp.zeros((L,), jnp.float32)
        acc2 = jnp.zeros((L,), jnp.float32)
        for d in range(L):
            col = jnp.full((L,), d, jnp.int32)
            a = plsc.load_gather(ra, [rows, col])
            b = plsc.load_gather(rb, [rows, col])
            acc1 = acc1 + a * b
            c = plsc.load_gather(rc, [rows, col])
            e = plsc.load_gather(rd, [rows, col])
            acc2 = acc2 + c * e
        o1[pl.ds(g * L, L)] = acc1
        o2[pl.ds(g * L, L)] = acc2
        return carry

    lax.fori_loop(0, bpw // L, group, 0)
    pltpu.sync_copy(o1, out1_hbm.at[pl.ds(base, bpw)])
    pltpu.sync_copy(o2, out2_hbm.at[pl.ds(base, bpw)])


def kernel(v_i, v_j, first_table, second_table, context_table):
    batch = v_i.shape[0]
    dim = first_table.shape[1]
    assert batch % (NW * L) == 0 and dim == L
    bpw = batch // NW
    v_i = v_i.astype(jnp.int32)
    v_j = v_j.astype(jnp.int32)

    mesh = plsc.VectorSubcoreMesh(core_axis_name="c", subcore_axis_name="s")
    f = pl.kernel(
        lambda *refs: _sc_body(bpw, *refs),
        out_type=(
            jax.ShapeDtypeStruct((batch,), jnp.float32),
            jax.ShapeDtypeStruct((batch,), jnp.float32),
        ),
        mesh=mesh,
        compiler_params=pltpu.CompilerParams(
            needs_layout_passes=False, use_tc_tiling_on_sc=False
        ),
        scratch_types=[
            pltpu.VMEM((bpw,), jnp.int32),
            pltpu.VMEM((bpw,), jnp.int32),
            pltpu.VMEM((bpw, dim), jnp.float32),
            pltpu.VMEM((bpw, dim), jnp.float32),
            pltpu.VMEM((bpw, dim), jnp.float32),
            pltpu.VMEM((bpw, dim), jnp.float32),
            pltpu.VMEM((bpw,), jnp.float32),
            pltpu.VMEM((bpw,), jnp.float32),
            pltpu.SemaphoreType.DMA,
        ],
    )
    first, second = f(v_i, v_j, first_table, second_table, context_table)
    return (first, second)
